# Initial kernel scaffold; baseline (speedup 1.0000x reference)
import jax
import jax.numpy as jnp
from jax.experimental import pallas as pl


def kernel(entity_emb, user_emb, edge_index, edge_type, user_edge_index, user_edge_type, mat_row, mat_col, weight, W1, W2, W3):
    raise NotImplementedError("write your pallas kernel here")


# speedup vs baseline: 1.0002x; 1.0002x over previous
import jax
import jax.numpy as jnp
from jax.experimental import pallas as pl

N_USERS = 50000
N_ITEMS = 50000
N_ENTITIES = 100000
N_USER_NODES = 100000


def _scatter_mean(src, index, dim_size):
    s = jax.ops.segment_sum(src, index, num_segments=dim_size)
    cnt = jax.ops.segment_sum(jnp.ones((src.shape[0],), src.dtype), index,
                              num_segments=dim_size)
    return s / jnp.maximum(cnt, 1.0)[:, None]


def kernel(entity_emb, user_emb, edge_index, edge_type, user_edge_index,
           user_edge_type, mat_row, mat_col, weight, W1, W2, W3):
    head, tail = edge_index[0], edge_index[1]
    neigh_relation_emb = entity_emb[tail] * weight[edge_type]
    entity_agg = _scatter_mean(neigh_relation_emb, head, N_ENTITIES)

    uhead, utail = user_edge_index[0], user_edge_index[1]
    uneigh = user_emb[utail] * weight[user_edge_type]
    attribute_agg = _scatter_mean(uneigh, uhead, N_USER_NODES)

    item_kg_agg = entity_agg[:N_ITEMS]
    item_neigh_emb = user_emb[mat_row] * weight[0]
    i_u_agg = _scatter_mean(item_neigh_emb, mat_col, N_ITEMS)
    gi = jax.nn.sigmoid(item_kg_agg @ W1.T + i_u_agg @ W2.T)
    item_emb_fusion = gi * item_kg_agg + (1.0 - gi) * i_u_agg

    user_ukg_agg = attribute_agg[:N_USERS]
    user_neigh_emb = entity_emb[mat_col] * weight[0]
    u_i_agg = _scatter_mean(user_neigh_emb, mat_row, N_USERS)
    hi = jax.nn.sigmoid(u_i_agg @ W2.T + user_ukg_agg @ W3.T)
    user_emb_fusion = hi * user_ukg_agg + (1.0 - hi) * u_i_agg

    user_agg = jnp.concatenate([user_emb_fusion, attribute_agg[N_USERS:]], axis=0)
    entity_agg_out = jnp.concatenate([item_emb_fusion, entity_agg[N_ITEMS:]], axis=0)
    return entity_agg_out, user_agg


# speedup vs baseline: 2.9220x; 2.9214x over previous
import functools

import jax
import jax.numpy as jnp
from jax import lax
from jax.experimental import pallas as pl
from jax.experimental.pallas import tpu as pltpu
from jax.experimental.pallas import tpu_sc as plsc

N_USERS = 50000
N_ITEMS = 50000
N_ENTITIES = 100000
N_USER_NODES = 100000

NC = 2
NS = 16
D = 64
F = 400
C = 25088
TRASH = 128
ACC = C + TRASH
STRIPE = C // NS


def _make_sc_gather(E):
    assert E % F == 0
    nblk = E // F
    mesh = plsc.VectorSubcoreMesh(core_axis_name="c", subcore_axis_name="s", num_cores=NC, num_subcores=NS)

    @functools.partial(
        pl.kernel, mesh=mesh,
        compiler_params=pltpu.CompilerParams(use_tc_tiling_on_sc=False),
        out_type=jax.ShapeDtypeStruct((E, D), jnp.float32),
        scratch_types=[
            pltpu.VMEM((F,), jnp.int32),
            pltpu.VMEM((F, D), jnp.float32),
            pltpu.SemaphoreType.DMA,
        ],
    )
    def k(table_hbm, idx_hbm, out_hbm, idx_v, rows_v, sem):
        wid = lax.axis_index("s") * NC + lax.axis_index("c")
        nw = NC * NS
        nmine = (nblk - wid + nw - 1) // nw

        def body(i, carry):
            base = (wid + i * nw) * F
            pltpu.sync_copy(idx_hbm.at[pl.ds(base, F)], idx_v)
            pltpu.async_copy(table_hbm.at[idx_v], rows_v, sem).wait()
            pltpu.sync_copy(rows_v, out_hbm.at[pl.ds(base, F)])
            return carry

        lax.fori_loop(0, nmine, body, 0)

    return k


def _make_sc_segsum(E, nchunk):
    assert E % F == 0
    nblk = E // F
    nrounds = nchunk // NC
    mesh = plsc.VectorSubcoreMesh(core_axis_name="c", subcore_axis_name="s", num_cores=NC, num_subcores=NS)

    @functools.partial(
        pl.kernel, mesh=mesh,
        compiler_params=pltpu.CompilerParams(use_tc_tiling_on_sc=False),
        out_type=(jax.ShapeDtypeStruct((nchunk * C, D), jnp.float32),
                  jax.ShapeDtypeStruct((nchunk * C,), jnp.float32)),
        scratch_types=[
            pltpu.VMEM((F,), jnp.int32),
            pltpu.VMEM((F,), jnp.int32),
            pltpu.VMEM((F, D), jnp.float32),
            pltpu.VMEM((F,), jnp.float32),
            pltpu.VMEM((F,), jnp.float32),
            pltpu.VMEM_SHARED((ACC, D), jnp.float32),
            pltpu.VMEM_SHARED((ACC,), jnp.float32),
            pltpu.SemaphoreType.DMA,
        ],
    )
    def k(vals_hbm, dst_hbm, z2_hbm, z1_hbm, ones_hbm, sums_hbm, cnts_hbm,
          idx_v, dloc_v, vals_v, ones_v, z1_v, acc_sh, cnt_sh, sem):
        cid = lax.axis_index("c")
        sid = lax.axis_index("s")
        lanes = lax.iota(jnp.int32, 16)

        pltpu.sync_copy(ones_hbm, ones_v)
        pltpu.sync_copy(z2_hbm, vals_v)
        pltpu.sync_copy(z1_hbm, z1_v)

        for r in range(nrounds):
            chunk = NC * r + cid
            lo = chunk * C

            zbase = sid * (ACC // NS)
            for t in range(ACC // NS // F):
                pltpu.sync_copy(vals_v, acc_sh.at[pl.ds(zbase + t * F, F)])
                pltpu.sync_copy(z1_v, cnt_sh.at[pl.ds(zbase + t * F, F)])
            rem = ACC // NS - (ACC // NS // F) * F
            if rem:
                off = zbase + (ACC // NS // F) * F
                pltpu.sync_copy(vals_v.at[pl.ds(0, rem)],
                                acc_sh.at[pl.ds(off, rem)])
                pltpu.sync_copy(z1_v.at[pl.ds(0, rem)],
                                cnt_sh.at[pl.ds(off, rem)])
            plsc.subcore_barrier()

            nmine = (nblk - sid + NS - 1) // NS

            def body(i, carry):
                base = (sid + i * NS) * F
                pltpu.sync_copy(dst_hbm.at[pl.ds(base, F)], idx_v)
                pltpu.sync_copy(vals_hbm.at[pl.ds(base, F)], vals_v)
                for j in range(F // 16):
                    d = idx_v[pl.ds(16 * j, 16)]
                    m = (d >= lo) & (d < lo + C)
                    tr = C + ((lanes + j) & (TRASH - 1))
                    dloc_v[pl.ds(16 * j, 16)] = jnp.where(m, d - lo, tr)
                pltpu.sync_copy(vals_v, acc_sh.at[dloc_v], add=True)
                pltpu.sync_copy(ones_v, cnt_sh.at[dloc_v], add=True)
                return carry

            lax.fori_loop(0, nmine, body, 0)
            plsc.subcore_barrier()

            for t in range(STRIPE // F):
                off = sid * STRIPE + t * F
                pltpu.sync_copy(acc_sh.at[pl.ds(off, F)], vals_v)
                pltpu.sync_copy(vals_v, sums_hbm.at[pl.ds(lo + off, F)])
                pltpu.sync_copy(cnt_sh.at[pl.ds(off, F)], z1_v)
                pltpu.sync_copy(z1_v, cnts_hbm.at[pl.ds(lo + off, F)])
            wrem = STRIPE - (STRIPE // F) * F
            if wrem:
                off = sid * STRIPE + (STRIPE // F) * F
                pltpu.sync_copy(acc_sh.at[pl.ds(off, wrem)],
                                vals_v.at[pl.ds(0, wrem)])
                pltpu.sync_copy(vals_v.at[pl.ds(0, wrem)],
                                sums_hbm.at[pl.ds(lo + off, wrem)])
                pltpu.sync_copy(cnt_sh.at[pl.ds(off, wrem)],
                                z1_v.at[pl.ds(0, wrem)])
                pltpu.sync_copy(z1_v.at[pl.ds(0, wrem)],
                                cnts_hbm.at[pl.ds(lo + off, wrem)])

            if r + 1 < nrounds:
                pltpu.sync_copy(z2_hbm, vals_v)
                pltpu.sync_copy(z1_hbm, z1_v)
                plsc.subcore_barrier()

    return k


def _tc_product(rows, types, weight):
    E = rows.shape[0]
    B = 1000
    assert E % B == 0

    def body(r_ref, t_ref, w_ref, o_ref):
        t = t_ref[...]
        oh = (t == lax.broadcasted_iota(jnp.int32, (B, 16), 1)
              ).astype(jnp.float32)
        wr = jnp.dot(oh, w_ref[...], preferred_element_type=jnp.float32)
        o_ref[...] = r_ref[...] * wr

    return pl.pallas_call(
        body,
        grid=(E // B,),
        in_specs=[pl.BlockSpec((B, D), lambda i: (i, 0)),
                  pl.BlockSpec((B, 1), lambda i: (i, 0)),
                  pl.BlockSpec((16, D), lambda i: (0, 0))],
        out_specs=pl.BlockSpec((B, D), lambda i: (i, 0)),
        out_shape=jax.ShapeDtypeStruct((E, D), jnp.float32),
    )(rows, types.reshape(E, 1), weight)


def _sigmoid(x):
    return 1.0 / (1.0 + jnp.exp(-x))


def _tc_gate(esum, ecnt, asum, acnt, iusum, iucnt, uisum, uicnt,
             weight, W1, W2, W3):
    B = 400
    nhalf = N_ITEMS // B

    def body(es_ref, ec_ref, as_ref, ac_ref, ius_ref, iuc_ref,
             uis_ref, uic_ref, w_ref, w1_ref, w2_ref, w3_ref,
             eo_ref, uo_ref):
        i = pl.program_id(0)
        ea = es_ref[...] / jnp.maximum(ec_ref[...], 1.0)
        ua = as_ref[...] / jnp.maximum(ac_ref[...], 1.0)

        @pl.when(i < nhalf)
        def _():
            w0 = w_ref[0:1, :]
            iu = (ius_ref[...] / jnp.maximum(iuc_ref[...], 1.0)) * w0
            ui = (uis_ref[...] / jnp.maximum(uic_ref[...], 1.0)) * w0
            dn = (((1,), (1,)), ((), ()))
            gi = _sigmoid(
                lax.dot_general(ea, w1_ref[...], dn,
                                preferred_element_type=jnp.float32)
                + lax.dot_general(iu, w2_ref[...], dn,
                                  preferred_element_type=jnp.float32))
            eo_ref[...] = gi * ea + (1.0 - gi) * iu
            hi = _sigmoid(
                lax.dot_general(ui, w2_ref[...], dn,
                                preferred_element_type=jnp.float32)
                + lax.dot_general(ua, w3_ref[...], dn,
                                  preferred_element_type=jnp.float32))
            uo_ref[...] = hi * ua + (1.0 - hi) * ui

        @pl.when(i >= nhalf)
        def _():
            eo_ref[...] = ea
            uo_ref[...] = ua

    row = lambda i: (i, 0)
    half = lambda i: (jnp.minimum(i, nhalf - 1), 0)
    full = lambda i: (0, 0)
    return pl.pallas_call(
        body,
        grid=(N_ENTITIES // B,),
        in_specs=[pl.BlockSpec((B, D), row), pl.BlockSpec((B, 1), row),
                  pl.BlockSpec((B, D), row), pl.BlockSpec((B, 1), row),
                  pl.BlockSpec((B, D), half), pl.BlockSpec((B, 1), half),
                  pl.BlockSpec((B, D), half), pl.BlockSpec((B, 1), half),
                  pl.BlockSpec((16, D), full), pl.BlockSpec((D, D), full),
                  pl.BlockSpec((D, D), full), pl.BlockSpec((D, D), full)],
        out_specs=[pl.BlockSpec((B, D), row), pl.BlockSpec((B, D), row)],
        out_shape=[jax.ShapeDtypeStruct((N_ENTITIES, D), jnp.float32),
                   jax.ShapeDtypeStruct((N_USER_NODES, D), jnp.float32)],
    )(esum, ecnt.reshape(-1, 1), asum, acnt.reshape(-1, 1),
      iusum, iucnt.reshape(-1, 1), uisum, uicnt.reshape(-1, 1),
      weight, W1, W2, W3)


def kernel(entity_emb, user_emb, edge_index, edge_type, user_edge_index,
           user_edge_type, mat_row, mat_col, weight, W1, W2, W3):
    E_KG = edge_index.shape[1]
    NNZ = mat_row.shape[0]
    head, tail = edge_index[0], edge_index[1]
    uhead, utail = user_edge_index[0], user_edge_index[1]

    gather_kg = _make_sc_gather(E_KG)
    gather_nz = _make_sc_gather(NNZ)
    segsum_kg = _make_sc_segsum(E_KG, 4)
    segsum_nz = _make_sc_segsum(NNZ, 2)

    z2 = jnp.zeros((F, D), jnp.float32)
    z1 = jnp.zeros((F,), jnp.float32)
    ones = jnp.ones((F,), jnp.float32)

    rows1 = gather_kg(entity_emb, tail)
    prod1 = _tc_product(rows1, edge_type, weight)
    esum, ecnt = segsum_kg(prod1, head, z2, z1, ones)

    rows2 = gather_kg(user_emb, utail)
    prod2 = _tc_product(rows2, user_edge_type, weight)
    asum, acnt = segsum_kg(prod2, uhead, z2, z1, ones)

    rows3 = gather_nz(user_emb, mat_row)
    iusum, iucnt = segsum_nz(rows3, mat_col, z2, z1, ones)

    rows4 = gather_nz(entity_emb, mat_col)
    uisum, uicnt = segsum_nz(rows4, mat_row, z2, z1, ones)

    return _tc_gate(esum, ecnt, asum, acnt, iusum, iucnt, uisum, uicnt,
                    weight, W1, W2, W3)


# speedup vs baseline: 3.0764x; 1.0528x over previous
import functools

import jax
import jax.numpy as jnp
from jax import lax
from jax.experimental import pallas as pl
from jax.experimental.pallas import tpu as pltpu
from jax.experimental.pallas import tpu_sc as plsc

N_USERS = 50000
N_ITEMS = 50000
N_ENTITIES = 100000
N_USER_NODES = 100000

NC = 2
NS = 16
D = 64
F = 400
TRASH = 128

C_NZ = 25088
ACC_NZ = C_NZ + TRASH
C_KG = 50048
ACC_KG = C_KG + TRASH

_MESH = plsc.VectorSubcoreMesh(core_axis_name="c", subcore_axis_name="s",
                               num_cores=NC, num_subcores=NS)
_NO_TC_TILING = pltpu.CompilerParams(use_tc_tiling_on_sc=False)


def _zero_stripe(sid, acc_sh, cnt_sh, z2_v, z1_v, acc_rows, with_cnt):
    stripe = acc_rows // NS
    zbase = sid * stripe
    for t in range(stripe // F):
        pltpu.sync_copy(z2_v, acc_sh.at[pl.ds(zbase + t * F, F)])
        if with_cnt:
            pltpu.sync_copy(z1_v, cnt_sh.at[pl.ds(zbase + t * F, F)])
    rem = stripe - (stripe // F) * F
    if rem:
        off = zbase + (stripe // F) * F
        pltpu.sync_copy(z2_v.at[pl.ds(0, rem)], acc_sh.at[pl.ds(off, rem)])
        if with_cnt:
            pltpu.sync_copy(z1_v.at[pl.ds(0, rem)], cnt_sh.at[pl.ds(off, rem)])


def _write_stripe(sid, lo, acc_sh, cnt_sh, sums_hbm, cnts_hbm,
                  stage2_v, stage1_v, rows, with_cnt):
    stripe = rows // NS
    nfull = stripe // F
    for t in range(nfull):
        off = sid * stripe + t * F
        pltpu.sync_copy(acc_sh.at[pl.ds(off, F)], stage2_v)
        pltpu.sync_copy(stage2_v, sums_hbm.at[pl.ds(lo + off, F)])
        if with_cnt:
            pltpu.sync_copy(cnt_sh.at[pl.ds(off, F)], stage1_v)
            pltpu.sync_copy(stage1_v, cnts_hbm.at[pl.ds(lo + off, F)])
    rem = stripe - nfull * F
    if rem:
        off = sid * stripe + nfull * F
        pltpu.sync_copy(acc_sh.at[pl.ds(off, rem)], stage2_v.at[pl.ds(0, rem)])
        pltpu.sync_copy(stage2_v.at[pl.ds(0, rem)],
                        sums_hbm.at[pl.ds(lo + off, rem)])
        if with_cnt:
            pltpu.sync_copy(cnt_sh.at[pl.ds(off, rem)],
                            stage1_v.at[pl.ds(0, rem)])
            pltpu.sync_copy(stage1_v.at[pl.ds(0, rem)],
                            cnts_hbm.at[pl.ds(lo + off, rem)])


def _make_sc_gather(E):
    assert E % F == 0
    nblk = E // F

    @functools.partial(
        pl.kernel, mesh=_MESH, compiler_params=_NO_TC_TILING,
        out_type=jax.ShapeDtypeStruct((E, D), jnp.float32),
        scratch_types=[
            pltpu.VMEM((F,), jnp.int32),
            pltpu.VMEM((F, D), jnp.float32),
            pltpu.SemaphoreType.DMA,
        ],
    )
    def k(table_hbm, idx_hbm, out_hbm, idx_v, rows_v, sem):
        wid = lax.axis_index("s") * NC + lax.axis_index("c")
        nw = NC * NS
        nmine = (nblk - wid + nw - 1) // nw

        def body(i, carry):
            base = (wid + i * nw) * F
            pltpu.sync_copy(idx_hbm.at[pl.ds(base, F)], idx_v)
            pltpu.async_copy(table_hbm.at[idx_v], rows_v, sem).wait()
            pltpu.sync_copy(rows_v, out_hbm.at[pl.ds(base, F)])
            return carry

        lax.fori_loop(0, nmine, body, 0)

    return k


def _make_sc_segsum_kg(E):
    assert E % F == 0
    nblk = E // F
    W = D // 2

    @functools.partial(
        pl.kernel, mesh=_MESH, compiler_params=_NO_TC_TILING,
        out_type=(jax.ShapeDtypeStruct((NC * C_KG, W), jnp.float32),
                  jax.ShapeDtypeStruct((NC * C_KG, W), jnp.float32),
                  jax.ShapeDtypeStruct((NC * C_KG,), jnp.float32)),
        scratch_types=[
            pltpu.VMEM((F,), jnp.int32),
            pltpu.VMEM((F,), jnp.int32),
            pltpu.VMEM((F, W), jnp.float32),
            pltpu.VMEM((F,), jnp.float32),
            pltpu.VMEM((F,), jnp.float32),
            pltpu.VMEM_SHARED((ACC_KG, W), jnp.float32),
            pltpu.VMEM_SHARED((ACC_KG,), jnp.float32),
            pltpu.SemaphoreType.DMA,
            pltpu.SemaphoreType.DMA,
        ],
    )
    def k(valsA_hbm, valsB_hbm, dst_hbm, z2_hbm, z1_hbm, ones_hbm,
          sumsA_hbm, sumsB_hbm, cnts_hbm,
          idx_v, dloc_v, vals_v, ones_v, z1_v, acc_sh, cnt_sh, sem, sem2):
        cid = lax.axis_index("c")
        sid = lax.axis_index("s")
        lanes = lax.iota(jnp.int32, 16)
        lo = cid * C_KG

        pltpu.sync_copy(ones_hbm, ones_v)
        pltpu.sync_copy(z2_hbm, vals_v)
        pltpu.sync_copy(z1_hbm, z1_v)

        for r, (vals_hbm, sums_hbm) in enumerate(
                ((valsA_hbm, sumsA_hbm), (valsB_hbm, sumsB_hbm))):
            _zero_stripe(sid, acc_sh, cnt_sh, vals_v, z1_v, ACC_KG, r == 0)
            plsc.subcore_barrier()

            nmine = (nblk - sid + NS - 1) // NS

            def body(i, carry):
                base = (sid + i * NS) * F
                cp = pltpu.async_copy(vals_hbm.at[pl.ds(base, F)],
                                      vals_v, sem2)
                pltpu.sync_copy(dst_hbm.at[pl.ds(base, F)], idx_v)
                for j in range(F // 16):
                    d = idx_v[pl.ds(16 * j, 16)]
                    m = (d >= lo) & (d < lo + C_KG)
                    tr = C_KG + ((lanes + j) & (TRASH - 1))
                    dloc_v[pl.ds(16 * j, 16)] = jnp.where(m, d - lo, tr)
                cp.wait()
                pltpu.sync_copy(vals_v, acc_sh.at[dloc_v], add=True)
                if r == 0:
                    pltpu.sync_copy(ones_v, cnt_sh.at[dloc_v], add=True)
                return carry

            lax.fori_loop(0, nmine, body, 0)
            plsc.subcore_barrier()

            _write_stripe(sid, lo, acc_sh, cnt_sh, sums_hbm, cnts_hbm,
                          vals_v, z1_v, C_KG, r == 0)

            if r == 0:
                pltpu.sync_copy(z2_hbm, vals_v)
                pltpu.sync_copy(z1_hbm, z1_v)
                plsc.subcore_barrier()

    return k


def _make_sc_segsum_gather(E, N):
    assert E % F == 0
    nblk = E // F

    @functools.partial(
        pl.kernel, mesh=_MESH, compiler_params=_NO_TC_TILING,
        out_type=(jax.ShapeDtypeStruct((NC * C_NZ, D), jnp.float32),
                  jax.ShapeDtypeStruct((NC * C_NZ,), jnp.float32)),
        scratch_types=[
            pltpu.VMEM((F,), jnp.int32),
            pltpu.VMEM((F,), jnp.int32),
            pltpu.VMEM((F,), jnp.int32),
            pltpu.VMEM((F, D), jnp.float32),
            pltpu.VMEM((F,), jnp.float32),
            pltpu.VMEM((F,), jnp.float32),
            pltpu.VMEM_SHARED((ACC_NZ, D), jnp.float32),
            pltpu.VMEM_SHARED((ACC_NZ,), jnp.float32),
            pltpu.SemaphoreType.DMA,
        ],
    )
    def k(table_hbm, src_hbm, dst_hbm, z2_hbm, z1_hbm, ones_hbm,
          sums_hbm, cnts_hbm,
          sidx_v, idx_v, dloc_v, vals_v, ones_v, z1_v, acc_sh, cnt_sh, sem):
        cid = lax.axis_index("c")
        sid = lax.axis_index("s")
        lanes = lax.iota(jnp.int32, 16)
        lo = cid * C_NZ

        pltpu.sync_copy(ones_hbm, ones_v)
        pltpu.sync_copy(z2_hbm, vals_v)
        pltpu.sync_copy(z1_hbm, z1_v)

        _zero_stripe(sid, acc_sh, cnt_sh, vals_v, z1_v, ACC_NZ, True)
        plsc.subcore_barrier()

        nmine = (nblk - sid + NS - 1) // NS

        def body(i, carry):
            base = (sid + i * NS) * F
            pltpu.sync_copy(src_hbm.at[pl.ds(base, F)], sidx_v)
            cp = pltpu.async_copy(table_hbm.at[sidx_v], vals_v, sem)
            pltpu.sync_copy(dst_hbm.at[pl.ds(base, F)], idx_v)
            for j in range(F // 16):
                d = idx_v[pl.ds(16 * j, 16)]
                m = (d >= lo) & (d < lo + C_NZ)
                tr = C_NZ + ((lanes + j) & (TRASH - 1))
                dloc_v[pl.ds(16 * j, 16)] = jnp.where(m, d - lo, tr)
            cp.wait()
            pltpu.sync_copy(vals_v, acc_sh.at[dloc_v], add=True)
            pltpu.sync_copy(ones_v, cnt_sh.at[dloc_v], add=True)
            return carry

        lax.fori_loop(0, nmine, body, 0)
        plsc.subcore_barrier()

        _write_stripe(sid, lo, acc_sh, cnt_sh, sums_hbm, cnts_hbm,
                      vals_v, z1_v, C_NZ, True)

    return k


def _tc_product(rows, types, weight):
    E = rows.shape[0]
    B = 1000
    assert E % B == 0

    def body(r_ref, t_ref, w_ref, oa_ref, ob_ref):
        t = t_ref[...]
        oh = (t == lax.broadcasted_iota(jnp.int32, (B, 16), 1)
              ).astype(jnp.float32)
        wr = jnp.dot(oh, w_ref[...], preferred_element_type=jnp.float32)
        prod = r_ref[...] * wr
        oa_ref[...] = prod[:, :D // 2]
        ob_ref[...] = prod[:, D // 2:]

    return pl.pallas_call(
        body,
        grid=(E // B,),
        in_specs=[pl.BlockSpec((B, D), lambda i: (i, 0)),
                  pl.BlockSpec((B, 1), lambda i: (i, 0)),
                  pl.BlockSpec((16, D), lambda i: (0, 0))],
        out_specs=[pl.BlockSpec((B, D // 2), lambda i: (i, 0)),
                   pl.BlockSpec((B, D // 2), lambda i: (i, 0))],
        out_shape=[jax.ShapeDtypeStruct((E, D // 2), jnp.float32),
                   jax.ShapeDtypeStruct((E, D // 2), jnp.float32)],
    )(rows, types.reshape(E, 1), weight)


def _sigmoid(x):
    return 1.0 / (1.0 + jnp.exp(-x))


def _tc_gate(esumA, esumB, ecnt, asumA, asumB, acnt,
             iusum, iucnt, uisum, uicnt, weight, W1, W2, W3):
    B = 400
    H = D // 2
    nhalf = N_ITEMS // B

    def body(esa_ref, esb_ref, ec_ref, asa_ref, asb_ref, ac_ref,
             ius_ref, iuc_ref, uis_ref, uic_ref,
             w_ref, w1_ref, w2_ref, w3_ref, eo_ref, uo_ref):
        i = pl.program_id(0)
        es = jnp.concatenate([esa_ref[...], esb_ref[...]], axis=1)
        asm = jnp.concatenate([asa_ref[...], asb_ref[...]], axis=1)
        ea = es / jnp.maximum(ec_ref[...], 1.0)
        ua = asm / jnp.maximum(ac_ref[...], 1.0)

        @pl.when(i < nhalf)
        def _():
            w0 = w_ref[0:1, :]
            iu = (ius_ref[...] / jnp.maximum(iuc_ref[...], 1.0)) * w0
            ui = (uis_ref[...] / jnp.maximum(uic_ref[...], 1.0)) * w0
            dn = (((1,), (1,)), ((), ()))
            gi = _sigmoid(
                lax.dot_general(ea, w1_ref[...], dn,
                                preferred_element_type=jnp.float32)
                + lax.dot_general(iu, w2_ref[...], dn,
                                  preferred_element_type=jnp.float32))
            eo_ref[...] = gi * ea + (1.0 - gi) * iu
            hi = _sigmoid(
                lax.dot_general(ui, w2_ref[...], dn,
                                preferred_element_type=jnp.float32)
                + lax.dot_general(ua, w3_ref[...], dn,
                                  preferred_element_type=jnp.float32))
            uo_ref[...] = hi * ua + (1.0 - hi) * ui

        @pl.when(i >= nhalf)
        def _():
            eo_ref[...] = ea
            uo_ref[...] = ua

    row = lambda i: (i, 0)
    half = lambda i: (jnp.minimum(i, nhalf - 1), 0)
    full = lambda i: (0, 0)
    return pl.pallas_call(
        body,
        grid=(N_ENTITIES // B,),
        in_specs=[pl.BlockSpec((B, H), row), pl.BlockSpec((B, H), row),
                  pl.BlockSpec((B, 1), row),
                  pl.BlockSpec((B, H), row), pl.BlockSpec((B, H), row),
                  pl.BlockSpec((B, 1), row),
                  pl.BlockSpec((B, D), half), pl.BlockSpec((B, 1), half),
                  pl.BlockSpec((B, D), half), pl.BlockSpec((B, 1), half),
                  pl.BlockSpec((16, D), full), pl.BlockSpec((D, D), full),
                  pl.BlockSpec((D, D), full), pl.BlockSpec((D, D), full)],
        out_specs=[pl.BlockSpec((B, D), row), pl.BlockSpec((B, D), row)],
        out_shape=[jax.ShapeDtypeStruct((N_ENTITIES, D), jnp.float32),
                   jax.ShapeDtypeStruct((N_USER_NODES, D), jnp.float32)],
    )(esumA, esumB, ecnt.reshape(-1, 1), asumA, asumB, acnt.reshape(-1, 1),
      iusum, iucnt.reshape(-1, 1), uisum, uicnt.reshape(-1, 1),
      weight, W1, W2, W3)


def kernel(entity_emb, user_emb, edge_index, edge_type, user_edge_index,
           user_edge_type, mat_row, mat_col, weight, W1, W2, W3):
    E_KG = edge_index.shape[1]
    NNZ = mat_row.shape[0]
    head, tail = edge_index[0], edge_index[1]
    uhead, utail = user_edge_index[0], user_edge_index[1]

    gather_kg = _make_sc_gather(E_KG)
    segsum_kg = _make_sc_segsum_kg(E_KG)
    segsum_nz_u = _make_sc_segsum_gather(NNZ, N_USER_NODES)
    segsum_nz_e = _make_sc_segsum_gather(NNZ, N_ENTITIES)

    z2 = jnp.zeros((F, D), jnp.float32)
    z2h = jnp.zeros((F, D // 2), jnp.float32)
    z1 = jnp.zeros((F,), jnp.float32)
    ones = jnp.ones((F,), jnp.float32)

    rows1 = gather_kg(entity_emb, tail)
    prod1A, prod1B = _tc_product(rows1, edge_type, weight)
    esumA, esumB, ecnt = segsum_kg(prod1A, prod1B, head, z2h, z1, ones)

    rows2 = gather_kg(user_emb, utail)
    prod2A, prod2B = _tc_product(rows2, user_edge_type, weight)
    asumA, asumB, acnt = segsum_kg(prod2A, prod2B, uhead, z2h, z1, ones)

    iusum, iucnt = segsum_nz_u(user_emb, mat_row, mat_col, z2, z1, ones)
    uisum, uicnt = segsum_nz_e(entity_emb, mat_col, mat_row, z2, z1, ones)

    return _tc_gate(esumA, esumB, ecnt, asumA, asumB, acnt,
                    iusum, iucnt, uisum, uicnt, weight, W1, W2, W3)
